# baseline (speedup 1.0000x reference)
import jax
import jax.numpy as jnp
from jax.experimental import pallas as pl
from jax.experimental.pallas import tpu as pltpu

_M = 256
_N = 256
_DIM = 256
_NROWS = _M * _N
_ROWS_PER_BLOCK = 4096
_G = _NROWS // _ROWS_PER_BLOCK


def _som_body(params_ref, x_ref, w_ref, out_ref, gmin_ref, gidx_ref, bmu_ref):
    i = pl.program_id(0)

    @pl.when(i == 0)
    def _init():
        gmin_ref[0] = jnp.float32(jnp.inf)
        gidx_ref[0] = jnp.int32(0)

    @pl.when(i < _G)
    def _phase1():
        w = w_ref[...]
        d = x_ref[...] - w + jnp.float32(1e-6)
        s2 = jnp.sum(d * d, axis=1, keepdims=True)
        m = jnp.min(s2)
        rows = jax.lax.broadcasted_iota(jnp.int32, (_ROWS_PER_BLOCK, 1), 0)
        idx = jnp.min(jnp.where(s2 == m, rows, _NROWS))

        @pl.when(m < gmin_ref[0])
        def _():
            gmin_ref[0] = m
            gidx_ref[0] = i * _ROWS_PER_BLOCK + idx

    @pl.when(i >= _G)
    def _phase2():
        @pl.when(i == _G)
        def _():
            bmu = gidx_ref[0]
            bmu_ref[0] = (bmu & 255).astype(jnp.float32)
            bmu_ref[1] = (bmu >> 8).astype(jnp.float32)

        b = 2 * _G - 1 - i
        alpha_op = params_ref[0]
        inv_sig2 = params_ref[1]
        rows = jax.lax.broadcasted_iota(jnp.int32, (_ROWS_PER_BLOCK, 1), 0) \
            + b * _ROWS_PER_BLOCK
        dx = (rows & 255).astype(jnp.float32) - bmu_ref[0]
        dy = (rows >> 8).astype(jnp.float32) - bmu_ref[1]
        s = alpha_op * jnp.exp(-(dx * dx + dy * dy) * inv_sig2)
        w = w_ref[...]
        out_ref[...] = w + s * (x_ref[...] - w)


def kernel(x, weights, it):
    itf = jnp.asarray(it, jnp.float32)
    lr = 1.0 - itf / 100.0
    alpha_op = jnp.float32(0.3) * lr
    sigma_op = jnp.float32(128.0) * lr
    inv_sig2 = 1.0 / (sigma_op * sigma_op)
    params = jnp.stack([alpha_op, inv_sig2])

    x2d = x.reshape(1, _DIM)

    def block_idx(i):
        return (jnp.where(i < _G, i, 2 * _G - 1 - i), 0)

    def out_idx(i):
        return (jnp.where(i < _G, _G - 1, 2 * _G - 1 - i), 0)

    return pl.pallas_call(
        _som_body,
        grid=(2 * _G,),
        in_specs=[
            pl.BlockSpec(memory_space=pltpu.SMEM),
            pl.BlockSpec((1, _DIM), lambda i: (0, 0)),
            pl.BlockSpec((_ROWS_PER_BLOCK, _DIM), block_idx),
        ],
        out_specs=pl.BlockSpec((_ROWS_PER_BLOCK, _DIM), out_idx),
        out_shape=jax.ShapeDtypeStruct((_NROWS, _DIM), jnp.float32),
        scratch_shapes=[
            pltpu.SMEM((1,), jnp.float32),
            pltpu.SMEM((1,), jnp.int32),
            pltpu.SMEM((2,), jnp.float32),
        ],
        compiler_params=pltpu.CompilerParams(
            dimension_semantics=("arbitrary",),
        ),
    )(params, x2d, weights)
